# initial kernel scaffold (unmeasured)
import jax
import jax.numpy as jnp
from jax import lax
from jax.experimental import pallas as pl
from jax.experimental.pallas import tpu as pltpu

N_DEV = 4
M, N = 4096, 8192
CHUNK_M = 256
N_CHUNKS = M // CHUNK_M


def _ar_body(x_ref, out_ref, recv1, recv2, send2,
             send_sem1, recv_sem1, send_sem2, recv_sem2,
             credit1, credit2):
    c = pl.program_id(0)
    nc = pl.num_programs(0)
    my = lax.axis_index("i")
    p1 = my ^ 1
    p2 = 3 - my

    barrier = pltpu.get_barrier_semaphore()

    @pl.when(c == 0)
    def _():
        pl.semaphore_signal(barrier, inc=1, device_id=(p1,),
                            device_id_type=pl.DeviceIdType.MESH)
        pl.semaphore_signal(barrier, inc=1, device_id=(p2,),
                            device_id_type=pl.DeviceIdType.MESH)
        pl.semaphore_wait(barrier, 2)

    @pl.when(c > 0)
    def _():
        pl.semaphore_wait(credit1, 1)

    rdma1 = pltpu.make_async_remote_copy(
        src_ref=x_ref, dst_ref=recv1,
        send_sem=send_sem1, recv_sem=recv_sem1,
        device_id=(p1,), device_id_type=pl.DeviceIdType.MESH)
    rdma1.start()
    rdma1.wait()
    send2[...] = x_ref[...] + recv1[...]

    @pl.when(c < nc - 1)
    def _():
        pl.semaphore_signal(credit1, inc=1, device_id=(p1,),
                            device_id_type=pl.DeviceIdType.MESH)

    @pl.when(c > 0)
    def _():
        pl.semaphore_wait(credit2, 1)

    rdma2 = pltpu.make_async_remote_copy(
        src_ref=send2, dst_ref=recv2,
        send_sem=send_sem2, recv_sem=recv_sem2,
        device_id=(p2,), device_id_type=pl.DeviceIdType.MESH)
    rdma2.start()
    rdma2.wait()
    out_ref[...] = send2[...] + recv2[...]

    @pl.when(c < nc - 1)
    def _():
        pl.semaphore_signal(credit2, inc=1, device_id=(p2,),
                            device_id_type=pl.DeviceIdType.MESH)


def _all_reduce(partial):
    return pl.pallas_call(
        _ar_body,
        grid=(N_CHUNKS,),
        in_specs=[pl.BlockSpec((CHUNK_M, N), lambda c: (c, 0))],
        out_specs=pl.BlockSpec((CHUNK_M, N), lambda c: (c, 0)),
        out_shape=jax.ShapeDtypeStruct((M, N), jnp.float32),
        scratch_shapes=[
            pltpu.VMEM((CHUNK_M, N), jnp.float32),
            pltpu.VMEM((CHUNK_M, N), jnp.float32),
            pltpu.VMEM((CHUNK_M, N), jnp.float32),
            pltpu.SemaphoreType.DMA,
            pltpu.SemaphoreType.DMA,
            pltpu.SemaphoreType.DMA,
            pltpu.SemaphoreType.DMA,
            pltpu.SemaphoreType.REGULAR,
            pltpu.SemaphoreType.REGULAR,
        ],
        compiler_params=pltpu.CompilerParams(collective_id=0),
    )(partial)


def kernel(x, w_mat):
    partial = jnp.dot(x, w_mat, preferred_element_type=jnp.float32)
    y = _all_reduce(partial)
    amax = jnp.max(jnp.abs(y))
    scale = amax / 448.0
    q = jnp.clip(y / scale, -448.0, 448.0)
    q = q.astype(jnp.float8_e4m3fn).astype(jnp.float32)
    return q * scale


# baseline (device time: 3622649 ns/iter reference)
import jax
import jax.numpy as jnp
from jax import lax
from jax.experimental import pallas as pl
from jax.experimental.pallas import tpu as pltpu

N_DEV = 4
M, N = 4096, 8192
CHUNK_M = 128
N_CHUNKS = M // CHUNK_M


def _ar_body(x_ref, out_ref, recv1, recv2, send2,
             send_sem1, recv_sem1, send_sem2, recv_sem2,
             credit1, credit2):
    c = pl.program_id(0)
    nc = pl.num_programs(0)
    my = lax.axis_index("i")
    p1 = my ^ 1
    p2 = 3 - my

    barrier = pltpu.get_barrier_semaphore()

    @pl.when(c == 0)
    def _():
        pl.semaphore_signal(barrier, inc=1, device_id=(p1,),
                            device_id_type=pl.DeviceIdType.MESH)
        pl.semaphore_signal(barrier, inc=1, device_id=(p2,),
                            device_id_type=pl.DeviceIdType.MESH)
        pl.semaphore_wait(barrier, 2)

    @pl.when(c > 0)
    def _():
        pl.semaphore_wait(credit1, 1)

    rdma1 = pltpu.make_async_remote_copy(
        src_ref=x_ref, dst_ref=recv1,
        send_sem=send_sem1, recv_sem=recv_sem1,
        device_id=(p1,), device_id_type=pl.DeviceIdType.MESH)
    rdma1.start()
    rdma1.wait()
    send2[...] = x_ref[...] + recv1[...]

    @pl.when(c < nc - 1)
    def _():
        pl.semaphore_signal(credit1, inc=1, device_id=(p1,),
                            device_id_type=pl.DeviceIdType.MESH)

    @pl.when(c > 0)
    def _():
        pl.semaphore_wait(credit2, 1)

    rdma2 = pltpu.make_async_remote_copy(
        src_ref=send2, dst_ref=recv2,
        send_sem=send_sem2, recv_sem=recv_sem2,
        device_id=(p2,), device_id_type=pl.DeviceIdType.MESH)
    rdma2.start()
    rdma2.wait()
    out_ref[...] = send2[...] + recv2[...]

    @pl.when(c < nc - 1)
    def _():
        pl.semaphore_signal(credit2, inc=1, device_id=(p2,),
                            device_id_type=pl.DeviceIdType.MESH)


def _all_reduce(partial):
    return pl.pallas_call(
        _ar_body,
        grid=(N_CHUNKS,),
        in_specs=[pl.BlockSpec((CHUNK_M, N), lambda c: (c, 0))],
        out_specs=pl.BlockSpec((CHUNK_M, N), lambda c: (c, 0)),
        out_shape=jax.ShapeDtypeStruct((M, N), jnp.float32),
        scratch_shapes=[
            pltpu.VMEM((CHUNK_M, N), jnp.float32),
            pltpu.VMEM((CHUNK_M, N), jnp.float32),
            pltpu.VMEM((CHUNK_M, N), jnp.float32),
            pltpu.SemaphoreType.DMA,
            pltpu.SemaphoreType.DMA,
            pltpu.SemaphoreType.DMA,
            pltpu.SemaphoreType.DMA,
            pltpu.SemaphoreType.REGULAR,
            pltpu.SemaphoreType.REGULAR,
        ],
        compiler_params=pltpu.CompilerParams(collective_id=0),
    )(partial)


def _snap_e4m3(v):
    a = jnp.abs(v)
    bits = lax.bitcast_convert_type(a, jnp.int32)
    biased = (bits >> 23) & 0xFF
    step_bits = jnp.where(a >= 2.0 ** -6, (biased - 3) << 23, (127 - 9) << 23)
    step = lax.bitcast_convert_type(step_bits.astype(jnp.int32), jnp.float32)
    snapped = jnp.minimum(jnp.round(a / step) * step, 448.0)
    return jnp.sign(v) * snapped


def kernel(x, w_mat):
    partial = jnp.dot(x, w_mat, preferred_element_type=jnp.float32,
                      precision=lax.Precision.HIGHEST)
    y = _all_reduce(partial)
    amax = jnp.max(jnp.abs(y))
    scale = amax / 448.0
    return _snap_e4m3(y / scale) * scale


# device time: 2185170 ns/iter; 1.6578x vs baseline; 1.6578x over previous
import jax
import jax.numpy as jnp
from jax import lax
from jax.experimental import pallas as pl
from jax.experimental.pallas import tpu as pltpu

N_DEV = 4
M, N = 4096, 8192
CHUNK_M = 128
N_CHUNKS = M // CHUNK_M


def _ar_body(x_ref, out_ref, recv1, recv2, send2,
             send_sem1, recv_sem1, send_sem2, recv_sem2,
             credit1, credit2):
    c = pl.program_id(0)
    my = lax.axis_index("i")
    p1 = my ^ 1
    p2 = 3 - my

    barrier = pltpu.get_barrier_semaphore()

    @pl.when(c == 0)
    def _():
        pl.semaphore_signal(barrier, inc=1, device_id=(p1,),
                            device_id_type=pl.DeviceIdType.MESH)
        pl.semaphore_signal(barrier, inc=1, device_id=(p2,),
                            device_id_type=pl.DeviceIdType.MESH)
        pl.semaphore_wait(barrier, 2)

    @pl.when(c < N_CHUNKS)
    def _():
        @pl.when(c > 0)
        def _():
            pl.semaphore_wait(credit1, 1)
        rdma1 = pltpu.make_async_remote_copy(
            src_ref=x_ref, dst_ref=recv1,
            send_sem=send_sem1, recv_sem=recv_sem1,
            device_id=(p1,), device_id_type=pl.DeviceIdType.MESH)
        rdma1.start()

    @pl.when(c > 0)
    def _():
        rdma2_prev = pltpu.make_async_remote_copy(
            src_ref=send2, dst_ref=recv2,
            send_sem=send_sem2, recv_sem=recv_sem2,
            device_id=(p2,), device_id_type=pl.DeviceIdType.MESH)
        rdma2_prev.wait()
        out_ref[...] = send2[...] + recv2[...]

        @pl.when(c < N_CHUNKS)
        def _():
            pl.semaphore_signal(credit2, inc=1, device_id=(p2,),
                                device_id_type=pl.DeviceIdType.MESH)

    @pl.when(c < N_CHUNKS)
    def _():
        rdma1 = pltpu.make_async_remote_copy(
            src_ref=x_ref, dst_ref=recv1,
            send_sem=send_sem1, recv_sem=recv_sem1,
            device_id=(p1,), device_id_type=pl.DeviceIdType.MESH)
        rdma1.wait()
        send2[...] = x_ref[...] + recv1[...]

        @pl.when(c < N_CHUNKS - 1)
        def _():
            pl.semaphore_signal(credit1, inc=1, device_id=(p1,),
                                device_id_type=pl.DeviceIdType.MESH)

        @pl.when(c > 0)
        def _():
            pl.semaphore_wait(credit2, 1)
        rdma2 = pltpu.make_async_remote_copy(
            src_ref=send2, dst_ref=recv2,
            send_sem=send_sem2, recv_sem=recv_sem2,
            device_id=(p2,), device_id_type=pl.DeviceIdType.MESH)
        rdma2.start()


def _all_reduce(partial):
    return pl.pallas_call(
        _ar_body,
        grid=(N_CHUNKS + 1,),
        in_specs=[pl.BlockSpec(
            (CHUNK_M, N), lambda c: (jnp.minimum(c, N_CHUNKS - 1), 0))],
        out_specs=pl.BlockSpec(
            (CHUNK_M, N), lambda c: (jnp.maximum(c - 1, 0), 0)),
        out_shape=jax.ShapeDtypeStruct((M, N), jnp.float32),
        scratch_shapes=[
            pltpu.VMEM((CHUNK_M, N), jnp.float32),
            pltpu.VMEM((CHUNK_M, N), jnp.float32),
            pltpu.VMEM((CHUNK_M, N), jnp.float32),
            pltpu.SemaphoreType.DMA,
            pltpu.SemaphoreType.DMA,
            pltpu.SemaphoreType.DMA,
            pltpu.SemaphoreType.DMA,
            pltpu.SemaphoreType.REGULAR,
            pltpu.SemaphoreType.REGULAR,
        ],
        compiler_params=pltpu.CompilerParams(collective_id=0),
    )(partial)


def _snap_e4m3(v):
    a = jnp.abs(v)
    bits = lax.bitcast_convert_type(a, jnp.int32)
    biased = (bits >> 23) & 0xFF
    step_bits = jnp.where(a >= 2.0 ** -6, (biased - 3) << 23, (127 - 9) << 23)
    step = lax.bitcast_convert_type(step_bits.astype(jnp.int32), jnp.float32)
    snapped = jnp.minimum(jnp.round(a / step) * step, 448.0)
    return jnp.sign(v) * snapped


def kernel(x, w_mat):
    partial = jnp.dot(x, w_mat, preferred_element_type=jnp.float32,
                      precision=lax.Precision.HIGHEST)
    y = _all_reduce(partial)
    amax = jnp.max(jnp.abs(y))
    scale = amax / 448.0
    return _snap_e4m3(y / scale) * scale
